# baseline (speedup 1.0000x reference)
import functools

import jax
import jax.numpy as jnp
from jax import lax
from jax.experimental import pallas as pl
from jax.experimental.pallas import tpu as pltpu
from jax.experimental.pallas import tpu_sc as plsc

N = 10000
E = 320000
G = 64
D = 128

NC = 2
NS = 16
NW = NC * NS
WIN = 128
WPS = 80
CH = 40
EPAD = NW * WPS * WIN
TRASH = N
NP = 10240

BR = 2048

_f32 = jnp.float32


def _sc_mesh():
    return plsc.VectorSubcoreMesh(core_axis_name="c", subcore_axis_name="s")


def _sc_deg(edges, zeros_init, ones_win):
    @functools.partial(
        pl.kernel,
        out_type=jax.ShapeDtypeStruct((NC, NP, D), _f32),
        mesh=_sc_mesh(),
        scratch_types=[
            pltpu.VMEM_SHARED((NP, D), _f32),
            pltpu.VMEM((WPS, WIN), jnp.int32),
            pltpu.VMEM((WIN, D), _f32),
        ],
    )
    def k(e_hbm, z_hbm, o_hbm, out_hbm, acc, idx_v, ones_v):
        c = lax.axis_index("c")
        s = lax.axis_index("s")
        wid = c * NS + s
        rz = NP // NS
        pltpu.sync_copy(z_hbm, acc.at[pl.ds(s * rz, rz)])
        pltpu.sync_copy(e_hbm.at[wid, 1, pl.ds(0, WPS)], idx_v)
        pltpu.sync_copy(o_hbm, ones_v)
        plsc.subcore_barrier()

        @pl.loop(0, WPS)
        def _(w):
            pltpu.sync_copy(ones_v, acc.at[idx_v.at[w]], add=True)

        plsc.subcore_barrier()
        pltpu.sync_copy(acc.at[pl.ds(s * rz, rz)],
                        out_hbm.at[c, pl.ds(s * rz, rz)])

    return k(edges, zeros_init, ones_win)


def _sc_agg(g, edges, W=D):
    @functools.partial(
        pl.kernel,
        out_type=jax.ShapeDtypeStruct((NC, NP, W), _f32),
        mesh=_sc_mesh(),
        scratch_types=[
            pltpu.VMEM_SHARED((NP, W), _f32),
            pltpu.VMEM((CH, WIN), jnp.int32),
            pltpu.VMEM((CH, WIN), jnp.int32),
            pltpu.VMEM((WIN, W), _f32),
            pltpu.VMEM((WIN, W), _f32),
            pltpu.SemaphoreType.DMA,
            pltpu.SemaphoreType.DMA,
            pltpu.SemaphoreType.DMA,
            pltpu.SemaphoreType.DMA,
        ],
    )
    def k(g_hbm, e_hbm, out_hbm, acc, src_v, dst_v, rows_a, rows_b,
          sem_a, sem_b, sem_sa, sem_sb):
        c = lax.axis_index("c")
        s = lax.axis_index("s")
        wid = c * NS + s
        ri = NP // NS
        pltpu.sync_copy(g_hbm.at[pl.ds(s * ri, ri)], acc.at[pl.ds(s * ri, ri)])
        plsc.subcore_barrier()

        def g_start(w, rows, sem):
            pltpu.make_async_copy(g_hbm.at[src_v.at[w]], rows, sem).start()

        def g_wait(w, rows, sem):
            pltpu.make_async_copy(g_hbm.at[src_v.at[w]], rows, sem).wait()

        def s_start(w, rows, sem):
            pltpu.make_async_copy(rows, acc.at[dst_v.at[w]], sem).start(add=True)

        def s_wait(w, rows, sem):
            pltpu.make_async_copy(rows, acc.at[dst_v.at[w]], sem).wait()

        for h in range(WPS // CH):
            pltpu.sync_copy(e_hbm.at[wid, 0, pl.ds(h * CH, CH)], src_v)
            pltpu.sync_copy(e_hbm.at[wid, 1, pl.ds(h * CH, CH)], dst_v)
            g_start(0, rows_a, sem_a)
            g_start(1, rows_b, sem_b)

            @pl.loop(0, CH - 2, step=2)
            def _(w):
                g_wait(w, rows_a, sem_a)
                s_start(w, rows_a, sem_sa)
                g_wait(w + 1, rows_b, sem_b)
                s_start(w + 1, rows_b, sem_sb)
                s_wait(w, rows_a, sem_sa)
                g_start(w + 2, rows_a, sem_a)
                s_wait(w + 1, rows_b, sem_sb)
                g_start(w + 3, rows_b, sem_b)

            g_wait(CH - 2, rows_a, sem_a)
            s_start(CH - 2, rows_a, sem_sa)
            g_wait(CH - 1, rows_b, sem_b)
            s_start(CH - 1, rows_b, sem_sb)
            s_wait(CH - 2, rows_a, sem_sa)
            s_wait(CH - 1, rows_b, sem_sb)

        plsc.subcore_barrier()
        pltpu.sync_copy(acc.at[pl.ds(s * ri, ri)],
                        out_hbm.at[c, pl.ds(s * ri, ri)])

    return k(g, edges)


def _dot(a, b):
    return lax.dot_general(a, b, (((1,), (0,)), ((), ())),
                           precision=lax.Precision.HIGHEST,
                           preferred_element_type=_f32)


def _tc_matmul(x, w):
    def body(x_ref, w_ref, o_ref):
        o_ref[...] = _dot(x_ref[...], w_ref[...])

    return pl.pallas_call(
        body,
        grid=(NP // BR,),
        in_specs=[pl.BlockSpec((BR, D), lambda i: (i, 0)),
                  pl.BlockSpec((D, D), lambda i: (0, 0))],
        out_specs=pl.BlockSpec((BR, D), lambda i: (i, 0)),
        out_shape=jax.ShapeDtypeStruct((NP, D), _f32),
    )(x, w)


def _tc_scale(h, degp):
    def body(h_ref, d_ref, g_ref, dinv_ref):
        deg = d_ref[0, :, 0] + d_ref[1, :, 0] + 1.0
        dinv = lax.rsqrt(deg)
        dinv_ref[...] = dinv[:, None]
        g_ref[...] = h_ref[...] * dinv[:, None]

    return pl.pallas_call(
        body,
        grid=(NP // BR,),
        in_specs=[pl.BlockSpec((BR, D), lambda i: (i, 0)),
                  pl.BlockSpec((NC, BR, D), lambda i: (0, i, 0))],
        out_specs=[pl.BlockSpec((BR, D), lambda i: (i, 0)),
                   pl.BlockSpec((BR, 1), lambda i: (i, 0))],
        out_shape=[jax.ShapeDtypeStruct((NP, D), _f32),
                   jax.ShapeDtypeStruct((NP, 1), _f32)],
    )(h, degp)


def _tc_layer(acc, g, dinv, b, w_next):
    def body(a_ref, g_ref, dinv_ref, b_ref, w_ref, o_ref):
        dinv = dinv_ref[...]
        z = (a_ref[0] + a_ref[1] - g_ref[...]) * dinv + b_ref[...]
        z = jnp.maximum(z, 0.0)
        o_ref[...] = _dot(z, w_ref[...]) * dinv

    return pl.pallas_call(
        body,
        grid=(NP // BR,),
        in_specs=[pl.BlockSpec((NC, BR, D), lambda i: (0, i, 0)),
                  pl.BlockSpec((BR, D), lambda i: (i, 0)),
                  pl.BlockSpec((BR, 1), lambda i: (i, 0)),
                  pl.BlockSpec((1, D), lambda i: (0, 0)),
                  pl.BlockSpec((D, D), lambda i: (0, 0))],
        out_specs=pl.BlockSpec((BR, D), lambda i: (i, 0)),
        out_shape=jax.ShapeDtypeStruct((NP, D), _f32),
    )(acc, g, dinv, b, w_next)


def _tc_final(acc, g, dinv, b, batch2d, wl, bl):
    def body(a_ref, g_ref, dinv_ref, b_ref, bat_ref, wl_ref, bl_ref, o_ref,
             pool_ref):
        i = pl.program_id(0)

        @pl.when(i == 0)
        def _():
            pool_ref[...] = jnp.zeros((G, D), _f32)

        dinv = dinv_ref[...]
        z = (a_ref[0] + a_ref[1] - g_ref[...]) * dinv + b_ref[...]
        z = jnp.maximum(z, 0.0)
        gids = lax.broadcasted_iota(jnp.int32, (1, G), 1)
        onehot = (bat_ref[...] == gids).astype(_f32)
        pool_ref[...] += lax.dot_general(
            onehot, z, (((0,), (0,)), ((), ())),
            precision=lax.Precision.HIGHEST, preferred_element_type=_f32)

        @pl.when(i == NP // BR - 1)
        def _():
            o_ref[...] = _dot(pool_ref[...], wl_ref[...]) + bl_ref[...]

    return pl.pallas_call(
        body,
        grid=(NP // BR,),
        in_specs=[pl.BlockSpec((NC, BR, D), lambda i: (0, i, 0)),
                  pl.BlockSpec((BR, D), lambda i: (i, 0)),
                  pl.BlockSpec((BR, 1), lambda i: (i, 0)),
                  pl.BlockSpec((1, D), lambda i: (0, 0)),
                  pl.BlockSpec((BR, 1), lambda i: (i, 0)),
                  pl.BlockSpec((D, D), lambda i: (0, 0)),
                  pl.BlockSpec((1, D), lambda i: (0, 0))],
        out_specs=pl.BlockSpec((G, D), lambda i: (0, 0)),
        out_shape=jax.ShapeDtypeStruct((G, D), _f32),
        scratch_shapes=[pltpu.VMEM((G, D), _f32)],
    )(acc, g, dinv, b, batch2d, wl, bl)


def kernel(x, edge_index, batch, W1, b1, W2, b2, Wl, bl):
    src = edge_index[0].astype(jnp.int32)
    dst = edge_index[1].astype(jnp.int32)
    src_p = jnp.concatenate([src, jnp.zeros((EPAD - E,), jnp.int32)])
    dst_p = jnp.concatenate([dst, jnp.full((EPAD - E,), TRASH, jnp.int32)])
    edges = jnp.stack([src_p.reshape(NW, WPS, WIN),
                       dst_p.reshape(NW, WPS, WIN)], axis=1)

    zeros_init = jnp.zeros((NP // NS, D), _f32)
    ones_win = jnp.ones((WIN, D), _f32)
    xp = jnp.concatenate([x, jnp.zeros((NP - N, D), _f32)])
    batch2d = jnp.concatenate([batch.astype(jnp.int32),
                               jnp.full((NP - N,), G, jnp.int32)]).reshape(NP, 1)
    b1r = b1.reshape(1, D)
    b2r = b2.reshape(1, D)
    blr = bl.reshape(1, D)

    degp = _sc_deg(edges, zeros_init, ones_win)
    h1 = _tc_matmul(xp, W1)
    g1, dinv = _tc_scale(h1, degp)
    acc1 = _sc_agg(g1, edges)
    g2 = _tc_layer(acc1, g1, dinv, b1r, W2)
    acc2 = _sc_agg(g2, edges)
    return _tc_final(acc2, g2, dinv, b2r, batch2d, Wl, blr)


# speedup vs baseline: 1.0289x; 1.0289x over previous
import functools

import jax
import jax.numpy as jnp
from jax import lax
from jax.experimental import pallas as pl
from jax.experimental.pallas import tpu as pltpu
from jax.experimental.pallas import tpu_sc as plsc

N = 10000
E = 320000
G = 64
D = 128

NC = 2
NS = 16
NW = NC * NS
WIN = 128
WPS = 80
CH = 40
EPAD = NW * WPS * WIN
TRASH = N
NP = 10240

BR = 2048

_f32 = jnp.float32


def _sc_mesh():
    return plsc.VectorSubcoreMesh(core_axis_name="c", subcore_axis_name="s")


def _sc_deg(edges, zeros_init, ones_win):
    @functools.partial(
        pl.kernel,
        out_type=jax.ShapeDtypeStruct((NC, NP, D), _f32),
        mesh=_sc_mesh(),
        scratch_types=[
            pltpu.VMEM_SHARED((NP, D), _f32),
            pltpu.VMEM((WPS, WIN), jnp.int32),
            pltpu.VMEM((WIN, D), _f32),
        ],
    )
    def k(e_hbm, z_hbm, o_hbm, out_hbm, acc, idx_v, ones_v):
        c = lax.axis_index("c")
        s = lax.axis_index("s")
        wid = c * NS + s
        rz = NP // NS
        pltpu.sync_copy(z_hbm, acc.at[pl.ds(s * rz, rz)])
        pltpu.sync_copy(e_hbm.at[wid, 1, pl.ds(0, WPS)], idx_v)
        pltpu.sync_copy(o_hbm, ones_v)
        plsc.subcore_barrier()

        @pl.loop(0, WPS)
        def _(w):
            pltpu.sync_copy(ones_v, acc.at[idx_v.at[w]], add=True)

        plsc.subcore_barrier()
        pltpu.sync_copy(acc.at[pl.ds(s * rz, rz)],
                        out_hbm.at[c, pl.ds(s * rz, rz)])

    return k(edges, zeros_init, ones_win)


def _sc_agg(g, edges, W=D):
    @functools.partial(
        pl.kernel,
        out_type=jax.ShapeDtypeStruct((NC, NP, W), _f32),
        mesh=_sc_mesh(),
        scratch_types=[
            pltpu.VMEM_SHARED((NP, W), _f32),
            pltpu.VMEM((CH, WIN), jnp.int32),
            pltpu.VMEM((CH, WIN), jnp.int32),
            pltpu.VMEM((WIN, W), _f32),
            pltpu.VMEM((WIN, W), _f32),
            pltpu.SemaphoreType.DMA,
            pltpu.SemaphoreType.DMA,
        ],
    )
    def k(g_hbm, e_hbm, out_hbm, acc, src_v, dst_v, rows_a, rows_b,
          sem_a, sem_b):
        c = lax.axis_index("c")
        s = lax.axis_index("s")
        wid = c * NS + s
        ri = NP // NS
        pltpu.sync_copy(g_hbm.at[pl.ds(s * ri, ri)], acc.at[pl.ds(s * ri, ri)])
        plsc.subcore_barrier()

        def start(w, rows, sem):
            pltpu.make_async_copy(g_hbm.at[src_v.at[w]], rows, sem).start()

        def finish(w, rows, sem):
            pltpu.make_async_copy(g_hbm.at[src_v.at[w]], rows, sem).wait()
            pltpu.sync_copy(rows, acc.at[dst_v.at[w]], add=True)

        for h in range(WPS // CH):
            pltpu.sync_copy(e_hbm.at[wid, 0, pl.ds(h * CH, CH)], src_v)
            pltpu.sync_copy(e_hbm.at[wid, 1, pl.ds(h * CH, CH)], dst_v)
            start(0, rows_a, sem_a)
            start(1, rows_b, sem_b)

            @pl.loop(0, CH - 2, step=2)
            def _(w):
                finish(w, rows_a, sem_a)
                start(w + 2, rows_a, sem_a)
                finish(w + 1, rows_b, sem_b)
                start(w + 3, rows_b, sem_b)

            finish(CH - 2, rows_a, sem_a)
            finish(CH - 1, rows_b, sem_b)

        plsc.subcore_barrier()
        pltpu.sync_copy(acc.at[pl.ds(s * ri, ri)],
                        out_hbm.at[c, pl.ds(s * ri, ri)])

    return k(g, edges)


def _dot(a, b):
    return lax.dot_general(a, b, (((1,), (0,)), ((), ())),
                           precision=lax.Precision.HIGHEST,
                           preferred_element_type=_f32)


def _tc_matmul(x, w):
    def body(x_ref, w_ref, o_ref):
        o_ref[...] = _dot(x_ref[...], w_ref[...])

    return pl.pallas_call(
        body,
        grid=(NP // BR,),
        in_specs=[pl.BlockSpec((BR, D), lambda i: (i, 0)),
                  pl.BlockSpec((D, D), lambda i: (0, 0))],
        out_specs=pl.BlockSpec((BR, D), lambda i: (i, 0)),
        out_shape=jax.ShapeDtypeStruct((NP, D), _f32),
    )(x, w)


def _tc_scale(h, degp):
    def body(h_ref, d_ref, g_ref, dinv_ref):
        deg = d_ref[0, :, 0] + d_ref[1, :, 0] + 1.0
        dinv = lax.rsqrt(deg)
        dinv_ref[...] = dinv[:, None]
        g_ref[...] = h_ref[...] * dinv[:, None]

    return pl.pallas_call(
        body,
        grid=(NP // BR,),
        in_specs=[pl.BlockSpec((BR, D), lambda i: (i, 0)),
                  pl.BlockSpec((NC, BR, D), lambda i: (0, i, 0))],
        out_specs=[pl.BlockSpec((BR, D), lambda i: (i, 0)),
                   pl.BlockSpec((BR, 1), lambda i: (i, 0))],
        out_shape=[jax.ShapeDtypeStruct((NP, D), _f32),
                   jax.ShapeDtypeStruct((NP, 1), _f32)],
    )(h, degp)


def _tc_layer(acc, g, dinv, b, w_next):
    def body(a_ref, g_ref, dinv_ref, b_ref, w_ref, o_ref):
        dinv = dinv_ref[...]
        z = (a_ref[0] + a_ref[1] - g_ref[...]) * dinv + b_ref[...]
        z = jnp.maximum(z, 0.0)
        o_ref[...] = _dot(z, w_ref[...]) * dinv

    return pl.pallas_call(
        body,
        grid=(NP // BR,),
        in_specs=[pl.BlockSpec((NC, BR, D), lambda i: (0, i, 0)),
                  pl.BlockSpec((BR, D), lambda i: (i, 0)),
                  pl.BlockSpec((BR, 1), lambda i: (i, 0)),
                  pl.BlockSpec((1, D), lambda i: (0, 0)),
                  pl.BlockSpec((D, D), lambda i: (0, 0))],
        out_specs=pl.BlockSpec((BR, D), lambda i: (i, 0)),
        out_shape=jax.ShapeDtypeStruct((NP, D), _f32),
    )(acc, g, dinv, b, w_next)


def _tc_final(acc, g, dinv, b, batch2d, wl, bl):
    def body(a_ref, g_ref, dinv_ref, b_ref, bat_ref, wl_ref, bl_ref, o_ref,
             pool_ref):
        i = pl.program_id(0)

        @pl.when(i == 0)
        def _():
            pool_ref[...] = jnp.zeros((G, D), _f32)

        dinv = dinv_ref[...]
        z = (a_ref[0] + a_ref[1] - g_ref[...]) * dinv + b_ref[...]
        z = jnp.maximum(z, 0.0)
        gids = lax.broadcasted_iota(jnp.int32, (1, G), 1)
        onehot = (bat_ref[...] == gids).astype(_f32)
        pool_ref[...] += lax.dot_general(
            onehot, z, (((0,), (0,)), ((), ())),
            precision=lax.Precision.HIGHEST, preferred_element_type=_f32)

        @pl.when(i == NP // BR - 1)
        def _():
            o_ref[...] = _dot(pool_ref[...], wl_ref[...]) + bl_ref[...]

    return pl.pallas_call(
        body,
        grid=(NP // BR,),
        in_specs=[pl.BlockSpec((NC, BR, D), lambda i: (0, i, 0)),
                  pl.BlockSpec((BR, D), lambda i: (i, 0)),
                  pl.BlockSpec((BR, 1), lambda i: (i, 0)),
                  pl.BlockSpec((1, D), lambda i: (0, 0)),
                  pl.BlockSpec((BR, 1), lambda i: (i, 0)),
                  pl.BlockSpec((D, D), lambda i: (0, 0)),
                  pl.BlockSpec((1, D), lambda i: (0, 0))],
        out_specs=pl.BlockSpec((G, D), lambda i: (0, 0)),
        out_shape=jax.ShapeDtypeStruct((G, D), _f32),
        scratch_shapes=[pltpu.VMEM((G, D), _f32)],
    )(acc, g, dinv, b, batch2d, wl, bl)


def kernel(x, edge_index, batch, W1, b1, W2, b2, Wl, bl):
    src = edge_index[0].astype(jnp.int32)
    dst = edge_index[1].astype(jnp.int32)
    src_p = jnp.concatenate([src, jnp.zeros((EPAD - E,), jnp.int32)])
    pad_dst = TRASH + (jnp.arange(EPAD - E, dtype=jnp.int32) % (NP - N))
    dst_p = jnp.concatenate([dst, pad_dst])
    edges = jnp.stack([src_p.reshape(NW, WPS, WIN),
                       dst_p.reshape(NW, WPS, WIN)], axis=1)

    zeros_init = jnp.zeros((NP // NS, D), _f32)
    ones_win = jnp.ones((WIN, D), _f32)
    xp = jnp.concatenate([x, jnp.zeros((NP - N, D), _f32)])
    batch2d = jnp.concatenate([batch.astype(jnp.int32),
                               jnp.full((NP - N,), G, jnp.int32)]).reshape(NP, 1)
    b1r = b1.reshape(1, D)
    b2r = b2.reshape(1, D)
    blr = bl.reshape(1, D)

    degp = _sc_deg(edges, zeros_init, ones_win)
    h1 = _tc_matmul(xp, W1)
    g1, dinv = _tc_scale(h1, degp)
    acc1 = _sc_agg(g1, edges)
    g2 = _tc_layer(acc1, g1, dinv, b1r, W2)
    acc2 = _sc_agg(g2, edges)
    return _tc_final(acc2, g2, dinv, b2r, batch2d, Wl, blr)


# speedup vs baseline: 2.9884x; 2.9046x over previous
import functools

import jax
import jax.numpy as jnp
from jax import lax
from jax.experimental import pallas as pl
from jax.experimental.pallas import tpu as pltpu
from jax.experimental.pallas import tpu_sc as plsc

N = 10000
E = 320000
G = 64
D = 128

NC = 2
NS = 16
NW = NC * NS
WIN = 128
WPS = 80
CH = 40
EPAD = NW * WPS * WIN
TRASH = N
NP = 10240

BR = 2048

_f32 = jnp.float32


def _sc_mesh():
    return plsc.VectorSubcoreMesh(core_axis_name="c", subcore_axis_name="s")


def _sc_deg(edges, zeros_init, ones_win):
    @functools.partial(
        pl.kernel,
        out_type=jax.ShapeDtypeStruct((NC, NP, D), _f32),
        mesh=_sc_mesh(),
        scratch_types=[
            pltpu.VMEM_SHARED((NP, D), _f32),
            pltpu.VMEM((WPS, WIN), jnp.int32),
            pltpu.VMEM((WIN, D), _f32),
        ],
    )
    def k(e_hbm, z_hbm, o_hbm, out_hbm, acc, idx_v, ones_v):
        c = lax.axis_index("c")
        s = lax.axis_index("s")
        wid = c * NS + s
        rz = NP // NS
        pltpu.sync_copy(z_hbm, acc.at[pl.ds(s * rz, rz)])
        pltpu.sync_copy(e_hbm.at[wid, 1, pl.ds(0, WPS)], idx_v)
        pltpu.sync_copy(o_hbm, ones_v)
        plsc.subcore_barrier()

        @pl.loop(0, WPS)
        def _(w):
            pltpu.sync_copy(ones_v, acc.at[idx_v.at[w]], add=True)

        plsc.subcore_barrier()
        pltpu.sync_copy(acc.at[pl.ds(s * rz, rz)],
                        out_hbm.at[c, pl.ds(s * rz, rz)])

    return k(edges, zeros_init, ones_win)


def _sc_agg(g, edges, W=D):
    @functools.partial(
        pl.kernel,
        out_type=jax.ShapeDtypeStruct((NC, NP, W), _f32),
        mesh=_sc_mesh(),
        scratch_types=[
            pltpu.VMEM_SHARED((NP, W), _f32),
            pltpu.VMEM((CH, WIN), jnp.int32),
            pltpu.VMEM((CH, WIN), jnp.int32),
            pltpu.VMEM((WIN, W), _f32),
            pltpu.VMEM((WIN, W), _f32),
            pltpu.SemaphoreType.DMA,
            pltpu.SemaphoreType.DMA,
        ],
    )
    def k(g_hbm, e_hbm, out_hbm, acc, src_v, dst_v, rows_a, rows_b,
          sem_a, sem_b):
        c = lax.axis_index("c")
        s = lax.axis_index("s")
        wid = c * NS + s
        ri = NP // NS
        pltpu.sync_copy(g_hbm.at[pl.ds(s * ri, ri)], acc.at[pl.ds(s * ri, ri)])
        plsc.subcore_barrier()

        def start(w, rows, sem):
            pltpu.make_async_copy(g_hbm.at[src_v.at[w]], rows, sem).start()

        def finish(w, rows, sem):
            pltpu.make_async_copy(g_hbm.at[src_v.at[w]], rows, sem).wait()
            pltpu.sync_copy(rows, acc.at[dst_v.at[w]], add=True)

        for h in range(WPS // CH):
            pltpu.sync_copy(e_hbm.at[wid, 0, pl.ds(h * CH, CH)], src_v)
            pltpu.sync_copy(e_hbm.at[wid, 1, pl.ds(h * CH, CH)], dst_v)
            start(0, rows_a, sem_a)
            start(1, rows_b, sem_b)

            @pl.loop(0, CH - 2, step=2)
            def _(w):
                finish(w, rows_a, sem_a)
                start(w + 2, rows_a, sem_a)
                finish(w + 1, rows_b, sem_b)
                start(w + 3, rows_b, sem_b)

            finish(CH - 2, rows_a, sem_a)
            finish(CH - 1, rows_b, sem_b)

        plsc.subcore_barrier()
        pltpu.sync_copy(acc.at[pl.ds(s * ri, ri)],
                        out_hbm.at[c, pl.ds(s * ri, ri)])

    return k(g, edges)


def _dot(a, b):
    return lax.dot_general(a, b, (((1,), (0,)), ((), ())),
                           precision=lax.Precision.HIGHEST,
                           preferred_element_type=_f32)


def _tc_matmul(x, w):
    def body(x_ref, w_ref, o_ref):
        o_ref[...] = _dot(x_ref[...], w_ref[...])

    return pl.pallas_call(
        body,
        grid=(NP // BR,),
        in_specs=[pl.BlockSpec((BR, D), lambda i: (i, 0)),
                  pl.BlockSpec((D, D), lambda i: (0, 0))],
        out_specs=pl.BlockSpec((BR, D), lambda i: (i, 0)),
        out_shape=jax.ShapeDtypeStruct((NP, D), _f32),
    )(x, w)


def _tc_scale(h, degp):
    def body(h_ref, d_ref, g_ref, dinv_ref):
        deg = d_ref[0, :, 0] + d_ref[1, :, 0] + 1.0
        dinv = lax.rsqrt(deg)
        dinv_ref[...] = dinv[:, None]
        g_ref[...] = h_ref[...] * dinv[:, None]

    return pl.pallas_call(
        body,
        grid=(NP // BR,),
        in_specs=[pl.BlockSpec((BR, D), lambda i: (i, 0)),
                  pl.BlockSpec((NC, BR, D), lambda i: (0, i, 0))],
        out_specs=[pl.BlockSpec((BR, D), lambda i: (i, 0)),
                   pl.BlockSpec((BR, 1), lambda i: (i, 0))],
        out_shape=[jax.ShapeDtypeStruct((NP, D), _f32),
                   jax.ShapeDtypeStruct((NP, 1), _f32)],
    )(h, degp)


def _tc_layer(acc, g, dinv, b, w_next):
    def body(a_ref, g_ref, dinv_ref, b_ref, w_ref, o_ref):
        dinv = dinv_ref[...]
        z = (a_ref[0] + a_ref[1] - g_ref[...]) * dinv + b_ref[...]
        z = jnp.maximum(z, 0.0)
        o_ref[...] = _dot(z, w_ref[...]) * dinv

    return pl.pallas_call(
        body,
        grid=(NP // BR,),
        in_specs=[pl.BlockSpec((NC, BR, D), lambda i: (0, i, 0)),
                  pl.BlockSpec((BR, D), lambda i: (i, 0)),
                  pl.BlockSpec((BR, 1), lambda i: (i, 0)),
                  pl.BlockSpec((1, D), lambda i: (0, 0)),
                  pl.BlockSpec((D, D), lambda i: (0, 0))],
        out_specs=pl.BlockSpec((BR, D), lambda i: (i, 0)),
        out_shape=jax.ShapeDtypeStruct((NP, D), _f32),
    )(acc, g, dinv, b, w_next)


def _tc_final(acc, g, dinv, b, batch2d, wl, bl):
    def body(a_ref, g_ref, dinv_ref, b_ref, bat_ref, wl_ref, bl_ref, o_ref,
             pool_ref):
        i = pl.program_id(0)

        @pl.when(i == 0)
        def _():
            pool_ref[...] = jnp.zeros((G, D), _f32)

        dinv = dinv_ref[...]
        z = (a_ref[0] + a_ref[1] - g_ref[...]) * dinv + b_ref[...]
        z = jnp.maximum(z, 0.0)
        gids = lax.broadcasted_iota(jnp.int32, (1, G), 1)
        onehot = (bat_ref[...] == gids).astype(_f32)
        pool_ref[...] += lax.dot_general(
            onehot, z, (((0,), (0,)), ((), ())),
            precision=lax.Precision.HIGHEST, preferred_element_type=_f32)

        @pl.when(i == NP // BR - 1)
        def _():
            o_ref[...] = _dot(pool_ref[...], wl_ref[...]) + bl_ref[...]

    return pl.pallas_call(
        body,
        grid=(NP // BR,),
        in_specs=[pl.BlockSpec((NC, BR, D), lambda i: (0, i, 0)),
                  pl.BlockSpec((BR, D), lambda i: (i, 0)),
                  pl.BlockSpec((BR, 1), lambda i: (i, 0)),
                  pl.BlockSpec((1, D), lambda i: (0, 0)),
                  pl.BlockSpec((BR, 1), lambda i: (i, 0)),
                  pl.BlockSpec((D, D), lambda i: (0, 0)),
                  pl.BlockSpec((1, D), lambda i: (0, 0))],
        out_specs=pl.BlockSpec((G, D), lambda i: (0, 0)),
        out_shape=jax.ShapeDtypeStruct((G, D), _f32),
        scratch_shapes=[pltpu.VMEM((G, D), _f32)],
    )(acc, g, dinv, b, batch2d, wl, bl)


def kernel(x, edge_index, batch, W1, b1, W2, b2, Wl, bl):
    src = edge_index[0].astype(jnp.int32)
    dst = edge_index[1].astype(jnp.int32)
    pad_src = jnp.arange(EPAD - E, dtype=jnp.int32) % N
    src_p = jnp.concatenate([src, pad_src])
    pad_dst = TRASH + (jnp.arange(EPAD - E, dtype=jnp.int32) % (NP - N))
    dst_p = jnp.concatenate([dst, pad_dst])
    edges = jnp.stack([src_p.reshape(NW, WPS, WIN),
                       dst_p.reshape(NW, WPS, WIN)], axis=1)

    zeros_init = jnp.zeros((NP // NS, D), _f32)
    ones_win = jnp.ones((WIN, D), _f32)
    xp = jnp.concatenate([x, jnp.zeros((NP - N, D), _f32)])
    batch2d = jnp.concatenate([batch.astype(jnp.int32),
                               jnp.full((NP - N,), G, jnp.int32)]).reshape(NP, 1)
    b1r = b1.reshape(1, D)
    b2r = b2.reshape(1, D)
    blr = bl.reshape(1, D)

    degp = _sc_deg(edges, zeros_init, ones_win)
    h1 = _tc_matmul(xp, W1)
    g1, dinv = _tc_scale(h1, degp)
    acc1 = _sc_agg(g1, edges)
    g2 = _tc_layer(acc1, g1, dinv, b1r, W2)
    acc2 = _sc_agg(g2, edges)
    return _tc_final(acc2, g2, dinv, b2r, batch2d, Wl, blr)


# speedup vs baseline: 3.6421x; 1.2187x over previous
import dataclasses
import functools

import jax
import jax.numpy as jnp
from jax import lax
from jax.experimental import pallas as pl
from jax.experimental.pallas import tpu as pltpu
from jax.experimental.pallas import tpu_sc as plsc

N = 10000
E = 320000
G = 64
D = 128

NC = 2
NS = 16
NW = NC * NS
WIN = 128
WPS = 80
CH = 40
EPAD = NW * WPS * WIN
TRASH = N
NP = 10240

BR = 2048

_f32 = jnp.float32


def _sc_mesh():
    return plsc.VectorSubcoreMesh(core_axis_name="c", subcore_axis_name="s")


def _sc_deg(edges, zeros_init, ones_win):
    @functools.partial(
        pl.kernel,
        out_type=jax.ShapeDtypeStruct((NC, NP, D), _f32),
        mesh=_sc_mesh(),
        scratch_types=[
            pltpu.VMEM_SHARED((NP, D), _f32),
            pltpu.VMEM((WPS, WIN), jnp.int32),
            pltpu.VMEM((WIN, D), _f32),
        ],
    )
    def k(e_hbm, z_hbm, o_hbm, out_hbm, acc, idx_v, ones_v):
        c = lax.axis_index("c")
        s = lax.axis_index("s")
        wid = c * NS + s
        rz = NP // NS
        pltpu.sync_copy(z_hbm, acc.at[pl.ds(s * rz, rz)])
        pltpu.sync_copy(e_hbm.at[wid, 1, pl.ds(0, WPS)], idx_v)
        pltpu.sync_copy(o_hbm, ones_v)
        plsc.subcore_barrier()

        @pl.loop(0, WPS)
        def _(w):
            pltpu.sync_copy(ones_v, acc.at[idx_v.at[w]], add=True)

        plsc.subcore_barrier()
        pltpu.sync_copy(acc.at[pl.ds(s * rz, rz)],
                        out_hbm.at[c, pl.ds(s * rz, rz)])

    return k(edges, zeros_init, ones_win)


def _sc_deg2(edges, zeros_np):
    cp = pltpu.CompilerParams()
    if "needs_layout_passes" in pltpu.CompilerParams.__dataclass_fields__:
        cp = dataclasses.replace(cp, needs_layout_passes=False)

    @functools.partial(
        pl.kernel,
        out_type=jax.ShapeDtypeStruct((NC, NP), _f32),
        mesh=_sc_mesh(),
        compiler_params=cp,
        scratch_types=[
            pltpu.VMEM_SHARED((NS, NP), _f32),
            pltpu.VMEM((NP,), _f32),
            pltpu.VMEM((WPS, WIN), jnp.int32),
            pltpu.VMEM((NS, NP // NS), _f32),
            pltpu.VMEM((NP // NS,), _f32),
        ],
    )
    def k(e_hbm, z_hbm, out_hbm, shared_h, hist, idx_v, part, outb):
        c = lax.axis_index("c")
        s = lax.axis_index("s")
        wid = c * NS + s
        rz = NP // NS
        pltpu.sync_copy(z_hbm, hist)
        pltpu.sync_copy(e_hbm.at[wid, 1, pl.ds(0, WPS)], idx_v)
        ones16 = jnp.full((16,), 1.0, _f32)

        @pl.loop(0, WPS)
        def _(w):
            @pl.loop(0, WIN // 16)
            def _(j):
                idx = idx_v[w, pl.ds(j * 16, 16)]
                plsc.addupdate_scatter(hist, [idx], ones16)

        pltpu.sync_copy(hist, shared_h.at[s])
        plsc.subcore_barrier()
        pltpu.sync_copy(shared_h.at[:, pl.ds(s * rz, rz)], part)

        @pl.loop(0, rz // 16)
        def _(kk):
            v = part[0, pl.ds(kk * 16, 16)]
            for r in range(1, NS):
                v = v + part[r, pl.ds(kk * 16, 16)]
            outb[pl.ds(kk * 16, 16)] = v

        pltpu.sync_copy(outb, out_hbm.at[c, pl.ds(s * rz, rz)])

    return k(edges, zeros_np)


def _sc_agg(g, edges, W=D):
    @functools.partial(
        pl.kernel,
        out_type=jax.ShapeDtypeStruct((NC, NP, W), _f32),
        mesh=_sc_mesh(),
        scratch_types=[
            pltpu.VMEM_SHARED((NP, W), _f32),
            pltpu.VMEM((CH, WIN), jnp.int32),
            pltpu.VMEM((CH, WIN), jnp.int32),
            pltpu.VMEM((WIN, W), _f32),
            pltpu.VMEM((WIN, W), _f32),
            pltpu.SemaphoreType.DMA,
            pltpu.SemaphoreType.DMA,
        ],
    )
    def k(g_hbm, e_hbm, out_hbm, acc, src_v, dst_v, rows_a, rows_b,
          sem_a, sem_b):
        c = lax.axis_index("c")
        s = lax.axis_index("s")
        wid = c * NS + s
        ri = NP // NS
        pltpu.sync_copy(g_hbm.at[pl.ds(s * ri, ri)], acc.at[pl.ds(s * ri, ri)])
        plsc.subcore_barrier()

        def start(w, rows, sem):
            pltpu.make_async_copy(g_hbm.at[src_v.at[w]], rows, sem).start()

        def finish(w, rows, sem):
            pltpu.make_async_copy(g_hbm.at[src_v.at[w]], rows, sem).wait()
            pltpu.sync_copy(rows, acc.at[dst_v.at[w]], add=True)

        for h in range(WPS // CH):
            pltpu.sync_copy(e_hbm.at[wid, 0, pl.ds(h * CH, CH)], src_v)
            pltpu.sync_copy(e_hbm.at[wid, 1, pl.ds(h * CH, CH)], dst_v)
            start(0, rows_a, sem_a)
            start(1, rows_b, sem_b)

            @pl.loop(0, CH - 2, step=2)
            def _(w):
                finish(w, rows_a, sem_a)
                start(w + 2, rows_a, sem_a)
                finish(w + 1, rows_b, sem_b)
                start(w + 3, rows_b, sem_b)

            finish(CH - 2, rows_a, sem_a)
            finish(CH - 1, rows_b, sem_b)

        plsc.subcore_barrier()
        pltpu.sync_copy(acc.at[pl.ds(s * ri, ri)],
                        out_hbm.at[c, pl.ds(s * ri, ri)])

    return k(g, edges)


def _dot(a, b):
    return lax.dot_general(a, b, (((1,), (0,)), ((), ())),
                           precision=lax.Precision.HIGHEST,
                           preferred_element_type=_f32)


def _tc_matmul(x, w):
    def body(x_ref, w_ref, o_ref):
        o_ref[...] = _dot(x_ref[...], w_ref[...])

    return pl.pallas_call(
        body,
        grid=(NP // BR,),
        in_specs=[pl.BlockSpec((BR, D), lambda i: (i, 0)),
                  pl.BlockSpec((D, D), lambda i: (0, 0))],
        out_specs=pl.BlockSpec((BR, D), lambda i: (i, 0)),
        out_shape=jax.ShapeDtypeStruct((NP, D), _f32),
    )(x, w)


def _tc_scale(h, degp):
    def body(h_ref, d_ref, g_ref, dinv_ref):
        deg = d_ref[0] + d_ref[1] + 1.0
        dinv = lax.rsqrt(deg)
        dinv_ref[...] = dinv[:, None]
        g_ref[...] = h_ref[...] * dinv[:, None]

    return pl.pallas_call(
        body,
        grid=(NP // BR,),
        in_specs=[pl.BlockSpec((BR, D), lambda i: (i, 0)),
                  pl.BlockSpec((NC, BR), lambda i: (0, i))],
        out_specs=[pl.BlockSpec((BR, D), lambda i: (i, 0)),
                   pl.BlockSpec((BR, 1), lambda i: (i, 0))],
        out_shape=[jax.ShapeDtypeStruct((NP, D), _f32),
                   jax.ShapeDtypeStruct((NP, 1), _f32)],
    )(h, degp)


def _tc_layer(acc, g, dinv, b, w_next):
    def body(a_ref, g_ref, dinv_ref, b_ref, w_ref, o_ref):
        dinv = dinv_ref[...]
        z = (a_ref[0] + a_ref[1] - g_ref[...]) * dinv + b_ref[...]
        z = jnp.maximum(z, 0.0)
        o_ref[...] = _dot(z, w_ref[...]) * dinv

    return pl.pallas_call(
        body,
        grid=(NP // BR,),
        in_specs=[pl.BlockSpec((NC, BR, D), lambda i: (0, i, 0)),
                  pl.BlockSpec((BR, D), lambda i: (i, 0)),
                  pl.BlockSpec((BR, 1), lambda i: (i, 0)),
                  pl.BlockSpec((1, D), lambda i: (0, 0)),
                  pl.BlockSpec((D, D), lambda i: (0, 0))],
        out_specs=pl.BlockSpec((BR, D), lambda i: (i, 0)),
        out_shape=jax.ShapeDtypeStruct((NP, D), _f32),
    )(acc, g, dinv, b, w_next)


def _tc_final(acc, g, dinv, b, batch2d, wl, bl):
    def body(a_ref, g_ref, dinv_ref, b_ref, bat_ref, wl_ref, bl_ref, o_ref,
             pool_ref):
        i = pl.program_id(0)

        @pl.when(i == 0)
        def _():
            pool_ref[...] = jnp.zeros((G, D), _f32)

        dinv = dinv_ref[...]
        z = (a_ref[0] + a_ref[1] - g_ref[...]) * dinv + b_ref[...]
        z = jnp.maximum(z, 0.0)
        gids = lax.broadcasted_iota(jnp.int32, (1, G), 1)
        onehot = (bat_ref[...] == gids).astype(_f32)
        pool_ref[...] += lax.dot_general(
            onehot, z, (((0,), (0,)), ((), ())),
            precision=lax.Precision.HIGHEST, preferred_element_type=_f32)

        @pl.when(i == NP // BR - 1)
        def _():
            o_ref[...] = _dot(pool_ref[...], wl_ref[...]) + bl_ref[...]

    return pl.pallas_call(
        body,
        grid=(NP // BR,),
        in_specs=[pl.BlockSpec((NC, BR, D), lambda i: (0, i, 0)),
                  pl.BlockSpec((BR, D), lambda i: (i, 0)),
                  pl.BlockSpec((BR, 1), lambda i: (i, 0)),
                  pl.BlockSpec((1, D), lambda i: (0, 0)),
                  pl.BlockSpec((BR, 1), lambda i: (i, 0)),
                  pl.BlockSpec((D, D), lambda i: (0, 0)),
                  pl.BlockSpec((1, D), lambda i: (0, 0))],
        out_specs=pl.BlockSpec((G, D), lambda i: (0, 0)),
        out_shape=jax.ShapeDtypeStruct((G, D), _f32),
        scratch_shapes=[pltpu.VMEM((G, D), _f32)],
    )(acc, g, dinv, b, batch2d, wl, bl)


def kernel(x, edge_index, batch, W1, b1, W2, b2, Wl, bl):
    src = edge_index[0].astype(jnp.int32)
    dst = edge_index[1].astype(jnp.int32)
    pad_src = jnp.arange(EPAD - E, dtype=jnp.int32) % N
    src_p = jnp.concatenate([src, pad_src])
    pad_dst = TRASH + (jnp.arange(EPAD - E, dtype=jnp.int32) % (NP - N))
    dst_p = jnp.concatenate([dst, pad_dst])
    edges = jnp.stack([src_p.reshape(NW, WPS, WIN),
                       dst_p.reshape(NW, WPS, WIN)], axis=1)

    zeros_np = jnp.zeros((NP,), _f32)
    xp = jnp.concatenate([x, jnp.zeros((NP - N, D), _f32)])
    batch2d = jnp.concatenate([batch.astype(jnp.int32),
                               jnp.full((NP - N,), G, jnp.int32)]).reshape(NP, 1)
    b1r = b1.reshape(1, D)
    b2r = b2.reshape(1, D)
    blr = bl.reshape(1, D)

    degp = _sc_deg2(edges, zeros_np)
    h1 = _tc_matmul(xp, W1)
    g1, dinv = _tc_scale(h1, degp)
    acc1 = _sc_agg(g1, edges)
    g2 = _tc_layer(acc1, g1, dinv, b1r, W2)
    acc2 = _sc_agg(g2, edges)
    return _tc_final(acc2, g2, dinv, b2r, batch2d, Wl, blr)


# speedup vs baseline: 3.7345x; 1.0254x over previous
import dataclasses
import functools

import jax
import jax.numpy as jnp
from jax import lax
from jax.experimental import pallas as pl
from jax.experimental.pallas import tpu as pltpu
from jax.experimental.pallas import tpu_sc as plsc

N = 10000
E = 320000
G = 64
D = 128

NC = 2
NS = 16
NW = NC * NS
WIN = 128
WPS = 80
CH = 40
EPAD = NW * WPS * WIN
TRASH = N
NP = 10240

BR = 2048

_f32 = jnp.float32


def _sc_mesh():
    return plsc.VectorSubcoreMesh(core_axis_name="c", subcore_axis_name="s")


def _sc_deg2(edges, zeros_np):
    cp = pltpu.CompilerParams()
    if "needs_layout_passes" in pltpu.CompilerParams.__dataclass_fields__:
        cp = dataclasses.replace(cp, needs_layout_passes=False)

    @functools.partial(
        pl.kernel,
        out_type=jax.ShapeDtypeStruct((NC, NP), _f32),
        mesh=_sc_mesh(),
        compiler_params=cp,
        scratch_types=[
            pltpu.VMEM_SHARED((NS, NP), _f32),
            pltpu.VMEM((NP,), _f32),
            pltpu.VMEM((WPS, WIN), jnp.int32),
            pltpu.VMEM((NS, NP // NS), _f32),
            pltpu.VMEM((NP // NS,), _f32),
        ],
    )
    def k(e_hbm, z_hbm, out_hbm, shared_h, hist, idx_v, part, outb):
        c = lax.axis_index("c")
        s = lax.axis_index("s")
        wid = c * NS + s
        rz = NP // NS
        pltpu.sync_copy(z_hbm, hist)
        pltpu.sync_copy(e_hbm.at[1, wid, pl.ds(0, WPS)], idx_v)
        ones16 = jnp.full((16,), 1.0, _f32)

        @pl.loop(0, WPS)
        def _(w):
            @pl.loop(0, WIN // 16)
            def _(j):
                idx = idx_v[w, pl.ds(j * 16, 16)]
                plsc.addupdate_scatter(hist, [idx], ones16)

        pltpu.sync_copy(hist, shared_h.at[s])
        plsc.subcore_barrier()
        pltpu.sync_copy(shared_h.at[:, pl.ds(s * rz, rz)], part)

        @pl.loop(0, rz // 16)
        def _(kk):
            v = part[0, pl.ds(kk * 16, 16)]
            for r in range(1, NS):
                v = v + part[r, pl.ds(kk * 16, 16)]
            outb[pl.ds(kk * 16, 16)] = v

        pltpu.sync_copy(outb, out_hbm.at[c, pl.ds(s * rz, rz)])

    return k(edges, zeros_np)


def _sc_agg(g, edges, W=D):
    @functools.partial(
        pl.kernel,
        out_type=jax.ShapeDtypeStruct((NC, NP, W), _f32),
        mesh=_sc_mesh(),
        scratch_types=[
            pltpu.VMEM_SHARED((NP, W), _f32),
            pltpu.VMEM((CH, WIN), jnp.int32),
            pltpu.VMEM((CH, WIN), jnp.int32),
            pltpu.VMEM((WIN, W), _f32),
            pltpu.VMEM((WIN, W), _f32),
            pltpu.SemaphoreType.DMA,
            pltpu.SemaphoreType.DMA,
        ],
    )
    def k(g_hbm, e_hbm, out_hbm, acc, src_v, dst_v, rows_a, rows_b,
          sem_a, sem_b):
        c = lax.axis_index("c")
        s = lax.axis_index("s")
        wid = c * NS + s
        ri = NP // NS
        pltpu.sync_copy(g_hbm.at[pl.ds(s * ri, ri)], acc.at[pl.ds(s * ri, ri)])
        plsc.subcore_barrier()

        def start(w, rows, sem):
            pltpu.make_async_copy(g_hbm.at[src_v.at[w]], rows, sem).start()

        def finish(w, rows, sem):
            pltpu.make_async_copy(g_hbm.at[src_v.at[w]], rows, sem).wait()
            pltpu.sync_copy(rows, acc.at[dst_v.at[w]], add=True)

        for h in range(WPS // CH):
            pltpu.sync_copy(e_hbm.at[0, wid, pl.ds(h * CH, CH)], src_v)
            pltpu.sync_copy(e_hbm.at[1, wid, pl.ds(h * CH, CH)], dst_v)
            start(0, rows_a, sem_a)
            start(1, rows_b, sem_b)

            @pl.loop(0, CH - 2, step=2)
            def _(w):
                finish(w, rows_a, sem_a)
                start(w + 2, rows_a, sem_a)
                finish(w + 1, rows_b, sem_b)
                start(w + 3, rows_b, sem_b)

            finish(CH - 2, rows_a, sem_a)
            finish(CH - 1, rows_b, sem_b)

        plsc.subcore_barrier()
        pltpu.sync_copy(acc.at[pl.ds(s * ri, ri)],
                        out_hbm.at[c, pl.ds(s * ri, ri)])

    return k(g, edges)


def _dot(a, b):
    return lax.dot_general(a, b, (((1,), (0,)), ((), ())),
                           precision=lax.Precision.HIGHEST,
                           preferred_element_type=_f32)


def _tc_matmul(x, w):
    def body(x_ref, w_ref, o_ref):
        o_ref[...] = _dot(x_ref[...], w_ref[...])

    return pl.pallas_call(
        body,
        grid=(NP // BR,),
        in_specs=[pl.BlockSpec((BR, D), lambda i: (i, 0)),
                  pl.BlockSpec((D, D), lambda i: (0, 0))],
        out_specs=pl.BlockSpec((BR, D), lambda i: (i, 0)),
        out_shape=jax.ShapeDtypeStruct((NP, D), _f32),
    )(x, w)


def _tc_scale(h, degp):
    def body(h_ref, d_ref, g_ref, dinv_ref):
        deg = d_ref[0] + d_ref[1] + 1.0
        dinv = lax.rsqrt(deg)
        dinv_ref[...] = dinv[:, None]
        g_ref[...] = h_ref[...] * dinv[:, None]

    return pl.pallas_call(
        body,
        grid=(NP // BR,),
        in_specs=[pl.BlockSpec((BR, D), lambda i: (i, 0)),
                  pl.BlockSpec((NC, BR), lambda i: (0, i))],
        out_specs=[pl.BlockSpec((BR, D), lambda i: (i, 0)),
                   pl.BlockSpec((BR, 1), lambda i: (i, 0))],
        out_shape=[jax.ShapeDtypeStruct((NP, D), _f32),
                   jax.ShapeDtypeStruct((NP, 1), _f32)],
    )(h, degp)


def _tc_layer(acc, g, dinv, b, w_next):
    def body(a_ref, g_ref, dinv_ref, b_ref, w_ref, o_ref):
        dinv = dinv_ref[...]
        z = (a_ref[0] + a_ref[1] - g_ref[...]) * dinv + b_ref[...]
        z = jnp.maximum(z, 0.0)
        o_ref[...] = _dot(z, w_ref[...]) * dinv

    return pl.pallas_call(
        body,
        grid=(NP // BR,),
        in_specs=[pl.BlockSpec((NC, BR, D), lambda i: (0, i, 0)),
                  pl.BlockSpec((BR, D), lambda i: (i, 0)),
                  pl.BlockSpec((BR, 1), lambda i: (i, 0)),
                  pl.BlockSpec((1, D), lambda i: (0, 0)),
                  pl.BlockSpec((D, D), lambda i: (0, 0))],
        out_specs=pl.BlockSpec((BR, D), lambda i: (i, 0)),
        out_shape=jax.ShapeDtypeStruct((NP, D), _f32),
    )(acc, g, dinv, b, w_next)


def _tc_final(acc, g, dinv, b, batch2d, wl, bl):
    def body(a_ref, g_ref, dinv_ref, b_ref, bat_ref, wl_ref, bl_ref, o_ref,
             pool_ref):
        i = pl.program_id(0)

        @pl.when(i == 0)
        def _():
            pool_ref[...] = jnp.zeros((G, D), _f32)

        dinv = dinv_ref[...]
        z = (a_ref[0] + a_ref[1] - g_ref[...]) * dinv + b_ref[...]
        z = jnp.maximum(z, 0.0)
        gids = lax.broadcasted_iota(jnp.int32, (1, G), 1)
        onehot = (bat_ref[...] == gids).astype(_f32)
        pool_ref[...] += lax.dot_general(
            onehot, z, (((0,), (0,)), ((), ())),
            precision=lax.Precision.HIGHEST, preferred_element_type=_f32)

        @pl.when(i == NP // BR - 1)
        def _():
            o_ref[...] = _dot(pool_ref[...], wl_ref[...]) + bl_ref[...]

    return pl.pallas_call(
        body,
        grid=(NP // BR,),
        in_specs=[pl.BlockSpec((NC, BR, D), lambda i: (0, i, 0)),
                  pl.BlockSpec((BR, D), lambda i: (i, 0)),
                  pl.BlockSpec((BR, 1), lambda i: (i, 0)),
                  pl.BlockSpec((1, D), lambda i: (0, 0)),
                  pl.BlockSpec((BR, 1), lambda i: (i, 0)),
                  pl.BlockSpec((D, D), lambda i: (0, 0)),
                  pl.BlockSpec((1, D), lambda i: (0, 0))],
        out_specs=pl.BlockSpec((G, D), lambda i: (0, 0)),
        out_shape=jax.ShapeDtypeStruct((G, D), _f32),
        scratch_shapes=[pltpu.VMEM((G, D), _f32)],
    )(acc, g, dinv, b, batch2d, wl, bl)


def kernel(x, edge_index, batch, W1, b1, W2, b2, Wl, bl):
    pad_src = jnp.arange(EPAD - E, dtype=jnp.int32) % N
    pad_dst = TRASH + (jnp.arange(EPAD - E, dtype=jnp.int32) % (NP - N))
    pad_blk = jnp.stack([pad_src, pad_dst])
    edges = jnp.concatenate([edge_index.astype(jnp.int32), pad_blk],
                            axis=1).reshape(2, NW, WPS, WIN)

    zeros_np = jnp.zeros((NP,), _f32)
    xp = jnp.concatenate([x, jnp.zeros((NP - N, D), _f32)])
    batch2d = jnp.concatenate([batch.astype(jnp.int32),
                               jnp.full((NP - N,), G, jnp.int32)]).reshape(NP, 1)
    b1r = b1.reshape(1, D)
    b2r = b2.reshape(1, D)
    blr = bl.reshape(1, D)

    degp = _sc_deg2(edges, zeros_np)
    h1 = _tc_matmul(xp, W1)
    g1, dinv = _tc_scale(h1, degp)
    acc1 = _sc_agg(g1, edges)
    g2 = _tc_layer(acc1, g1, dinv, b1r, W2)
    acc2 = _sc_agg(g2, edges)
    return _tc_final(acc2, g2, dinv, b2r, batch2d, Wl, blr)


# speedup vs baseline: 3.8528x; 1.0317x over previous
import dataclasses
import functools

import jax
import jax.numpy as jnp
from jax import lax
from jax.experimental import pallas as pl
from jax.experimental.pallas import tpu as pltpu
from jax.experimental.pallas import tpu_sc as plsc

N = 10000
E = 320000
G = 64
D = 128

NC = 2
NS = 16
NW = NC * NS
WIN = 64
WPS = 160
CH = 40
NBUF = 4
EPAD = NW * WPS * WIN
TRASH = N
NP = 10240

BR = 2048

_f32 = jnp.float32


def _sc_mesh():
    return plsc.VectorSubcoreMesh(core_axis_name="c", subcore_axis_name="s")


def _sc_deg2(edges, zeros_np):
    cp = pltpu.CompilerParams()
    if "needs_layout_passes" in pltpu.CompilerParams.__dataclass_fields__:
        cp = dataclasses.replace(cp, needs_layout_passes=False)

    @functools.partial(
        pl.kernel,
        out_type=jax.ShapeDtypeStruct((NC, NP), _f32),
        mesh=_sc_mesh(),
        compiler_params=cp,
        scratch_types=[
            pltpu.VMEM_SHARED((NS, NP), _f32),
            pltpu.VMEM((NP,), _f32),
            pltpu.VMEM((WPS, WIN), jnp.int32),
            pltpu.VMEM((NS, NP // NS), _f32),
            pltpu.VMEM((NP // NS,), _f32),
        ],
    )
    def k(e_hbm, z_hbm, out_hbm, shared_h, hist, idx_v, part, outb):
        c = lax.axis_index("c")
        s = lax.axis_index("s")
        wid = c * NS + s
        rz = NP // NS
        pltpu.sync_copy(z_hbm, hist)
        pltpu.sync_copy(e_hbm.at[1, wid, pl.ds(0, WPS)], idx_v)
        ones16 = jnp.full((16,), 1.0, _f32)

        @pl.loop(0, WPS)
        def _(w):
            @pl.loop(0, WIN // 16)
            def _(j):
                idx = idx_v[w, pl.ds(j * 16, 16)]
                plsc.addupdate_scatter(hist, [idx], ones16)

        pltpu.sync_copy(hist, shared_h.at[s])
        plsc.subcore_barrier()
        pltpu.sync_copy(shared_h.at[:, pl.ds(s * rz, rz)], part)

        @pl.loop(0, rz // 16)
        def _(kk):
            v = part[0, pl.ds(kk * 16, 16)]
            for r in range(1, NS):
                v = v + part[r, pl.ds(kk * 16, 16)]
            outb[pl.ds(kk * 16, 16)] = v

        pltpu.sync_copy(outb, out_hbm.at[c, pl.ds(s * rz, rz)])

    return k(edges, zeros_np)


def _sc_agg(g, edges, W=D):
    @functools.partial(
        pl.kernel,
        out_type=jax.ShapeDtypeStruct((NC, NP, W), _f32),
        mesh=_sc_mesh(),
        scratch_types=[
            pltpu.VMEM_SHARED((NP, W), _f32),
            pltpu.VMEM((CH, WIN), jnp.int32),
            pltpu.VMEM((CH, WIN), jnp.int32),
        ] + [pltpu.VMEM((WIN, W), _f32)] * NBUF
          + [pltpu.SemaphoreType.DMA] * NBUF,
    )
    def k(g_hbm, e_hbm, out_hbm, acc, src_v, dst_v, *bufs_and_sems):
        rows_bufs = bufs_and_sems[:NBUF]
        sems = bufs_and_sems[NBUF:]
        c = lax.axis_index("c")
        s = lax.axis_index("s")
        wid = c * NS + s
        ri = NP // NS
        pltpu.sync_copy(g_hbm.at[pl.ds(s * ri, ri)], acc.at[pl.ds(s * ri, ri)])
        plsc.subcore_barrier()

        def start(w, rows, sem):
            pltpu.make_async_copy(g_hbm.at[src_v.at[w]], rows, sem).start()

        def finish(w, rows, sem):
            pltpu.make_async_copy(g_hbm.at[src_v.at[w]], rows, sem).wait()
            pltpu.sync_copy(rows, acc.at[dst_v.at[w]], add=True)

        for h in range(WPS // CH):
            pltpu.sync_copy(e_hbm.at[0, wid, pl.ds(h * CH, CH)], src_v)
            pltpu.sync_copy(e_hbm.at[1, wid, pl.ds(h * CH, CH)], dst_v)
            for i in range(NBUF):
                start(i, rows_bufs[i], sems[i])

            @pl.loop(0, CH - NBUF, step=NBUF)
            def _(w):
                for i in range(NBUF):
                    finish(w + i, rows_bufs[i], sems[i])
                    start(w + NBUF + i, rows_bufs[i], sems[i])

            for i in range(NBUF):
                finish(CH - NBUF + i, rows_bufs[i], sems[i])

        plsc.subcore_barrier()
        pltpu.sync_copy(acc.at[pl.ds(s * ri, ri)],
                        out_hbm.at[c, pl.ds(s * ri, ri)])

    return k(g, edges)


def _dot(a, b):
    return lax.dot_general(a, b, (((1,), (0,)), ((), ())),
                           precision=lax.Precision.HIGHEST,
                           preferred_element_type=_f32)


def _tc_matmul(x, w):
    def body(x_ref, w_ref, o_ref):
        o_ref[...] = _dot(x_ref[...], w_ref[...])

    return pl.pallas_call(
        body,
        grid=(NP // BR,),
        in_specs=[pl.BlockSpec((BR, D), lambda i: (i, 0)),
                  pl.BlockSpec((D, D), lambda i: (0, 0))],
        out_specs=pl.BlockSpec((BR, D), lambda i: (i, 0)),
        out_shape=jax.ShapeDtypeStruct((NP, D), _f32),
    )(x, w)


def _tc_scale(h, degp):
    def body(h_ref, d_ref, g_ref, dinv_ref):
        deg = d_ref[0] + d_ref[1] + 1.0
        dinv = lax.rsqrt(deg)
        dinv_ref[...] = dinv[:, None]
        g_ref[...] = h_ref[...] * dinv[:, None]

    return pl.pallas_call(
        body,
        grid=(NP // BR,),
        in_specs=[pl.BlockSpec((BR, D), lambda i: (i, 0)),
                  pl.BlockSpec((NC, BR), lambda i: (0, i))],
        out_specs=[pl.BlockSpec((BR, D), lambda i: (i, 0)),
                   pl.BlockSpec((BR, 1), lambda i: (i, 0))],
        out_shape=[jax.ShapeDtypeStruct((NP, D), _f32),
                   jax.ShapeDtypeStruct((NP, 1), _f32)],
    )(h, degp)


def _tc_layer(acc, g, dinv, b, w_next):
    def body(a_ref, g_ref, dinv_ref, b_ref, w_ref, o_ref):
        dinv = dinv_ref[...]
        z = (a_ref[0] + a_ref[1] - g_ref[...]) * dinv + b_ref[...]
        z = jnp.maximum(z, 0.0)
        o_ref[...] = _dot(z, w_ref[...]) * dinv

    return pl.pallas_call(
        body,
        grid=(NP // BR,),
        in_specs=[pl.BlockSpec((NC, BR, D), lambda i: (0, i, 0)),
                  pl.BlockSpec((BR, D), lambda i: (i, 0)),
                  pl.BlockSpec((BR, 1), lambda i: (i, 0)),
                  pl.BlockSpec((1, D), lambda i: (0, 0)),
                  pl.BlockSpec((D, D), lambda i: (0, 0))],
        out_specs=pl.BlockSpec((BR, D), lambda i: (i, 0)),
        out_shape=jax.ShapeDtypeStruct((NP, D), _f32),
    )(acc, g, dinv, b, w_next)


def _tc_final(acc, g, dinv, b, batch2d, wl, bl):
    def body(a_ref, g_ref, dinv_ref, b_ref, bat_ref, wl_ref, bl_ref, o_ref,
             pool_ref):
        i = pl.program_id(0)

        @pl.when(i == 0)
        def _():
            pool_ref[...] = jnp.zeros((G, D), _f32)

        dinv = dinv_ref[...]
        z = (a_ref[0] + a_ref[1] - g_ref[...]) * dinv + b_ref[...]
        z = jnp.maximum(z, 0.0)
        gids = lax.broadcasted_iota(jnp.int32, (1, G), 1)
        onehot = (bat_ref[...] == gids).astype(_f32)
        pool_ref[...] += lax.dot_general(
            onehot, z, (((0,), (0,)), ((), ())),
            precision=lax.Precision.HIGHEST, preferred_element_type=_f32)

        @pl.when(i == NP // BR - 1)
        def _():
            o_ref[...] = _dot(pool_ref[...], wl_ref[...]) + bl_ref[...]

    return pl.pallas_call(
        body,
        grid=(NP // BR,),
        in_specs=[pl.BlockSpec((NC, BR, D), lambda i: (0, i, 0)),
                  pl.BlockSpec((BR, D), lambda i: (i, 0)),
                  pl.BlockSpec((BR, 1), lambda i: (i, 0)),
                  pl.BlockSpec((1, D), lambda i: (0, 0)),
                  pl.BlockSpec((BR, 1), lambda i: (i, 0)),
                  pl.BlockSpec((D, D), lambda i: (0, 0)),
                  pl.BlockSpec((1, D), lambda i: (0, 0))],
        out_specs=pl.BlockSpec((G, D), lambda i: (0, 0)),
        out_shape=jax.ShapeDtypeStruct((G, D), _f32),
        scratch_shapes=[pltpu.VMEM((G, D), _f32)],
    )(acc, g, dinv, b, batch2d, wl, bl)


def kernel(x, edge_index, batch, W1, b1, W2, b2, Wl, bl):
    pad_src = jnp.arange(EPAD - E, dtype=jnp.int32) % N
    pad_dst = TRASH + (jnp.arange(EPAD - E, dtype=jnp.int32) % (NP - N))
    pad_blk = jnp.stack([pad_src, pad_dst])
    edges = jnp.concatenate([edge_index.astype(jnp.int32), pad_blk],
                            axis=1).reshape(2, NW, WPS, WIN)

    zeros_np = jnp.zeros((NP,), _f32)
    xp = jnp.concatenate([x, jnp.zeros((NP - N, D), _f32)])
    batch2d = jnp.concatenate([batch.astype(jnp.int32),
                               jnp.full((NP - N,), G, jnp.int32)]).reshape(NP, 1)
    b1r = b1.reshape(1, D)
    b2r = b2.reshape(1, D)
    blr = bl.reshape(1, D)

    degp = _sc_deg2(edges, zeros_np)
    h1 = _tc_matmul(xp, W1)
    g1, dinv = _tc_scale(h1, degp)
    acc1 = _sc_agg(g1, edges)
    g2 = _tc_layer(acc1, g1, dinv, b1r, W2)
    acc2 = _sc_agg(g2, edges)
    return _tc_final(acc2, g2, dinv, b2r, batch2d, Wl, blr)
